# baseline (speedup 1.0000x reference)
import functools

import jax
import jax.numpy as jnp
from jax import lax
from jax.experimental import pallas as pl
from jax.experimental.pallas import tpu as pltpu
from jax.experimental.pallas import tpu_sc as plsc

_CHUNKS_PER_CORE = 16


def kernel(table, seq_len):
    n, d = table.shape
    info = plsc.get_sparse_core_info()
    num_cores = info.num_cores
    rows_per_c = n // num_cores
    chunk = rows_per_c // _CHUNKS_PER_CORE

    slen_arr = jnp.reshape(jnp.asarray(seq_len, jnp.int32), (1,))

    mesh = plsc.ScalarSubcoreMesh(axis_name="c", num_cores=num_cores)

    @functools.partial(
        pl.kernel,
        mesh=mesh,
        out_type=jax.ShapeDtypeStruct((n, d), table.dtype),
        scratch_types=[
            pltpu.SMEM((1,), jnp.int32),
            pltpu.VMEM_SHARED((chunk, 128), jnp.float32),
            pltpu.VMEM_SHARED((1, 128), jnp.float32),
            pltpu.SemaphoreType.DMA,
        ],
    )
    def scs(table_hbm, slen_hbm, out_hbm, slen_s, stage, lrow, sem):
        cid = lax.axis_index("c")
        base = cid * rows_per_c
        pltpu.sync_copy(slen_hbm, slen_s)
        s = slen_s[0]
        for j in range(_CHUNKS_PER_CORE):
            start = base + j * chunk

            @pl.when(start + chunk <= s)
            def _bulk():
                pltpu.sync_copy(table_hbm.at[pl.ds(start, chunk)], stage)
                pltpu.sync_copy(stage, out_hbm.at[pl.ds(start, chunk)])

            @pl.when(start + chunk > s)
            def _mixed():
                pltpu.sync_copy(table_hbm.at[pl.ds(s - 1, 1)], lrow)

                def body(k, _):
                    row = start + k

                    @pl.when(row < s)
                    def _id():
                        pltpu.sync_copy(table_hbm.at[pl.ds(row, 1)],
                                        stage.at[pl.ds(0, 1)])
                        pltpu.sync_copy(stage.at[pl.ds(0, 1)],
                                        out_hbm.at[pl.ds(row, 1)])

                    @pl.when(row >= s)
                    def _fill():
                        pltpu.sync_copy(lrow, out_hbm.at[pl.ds(row, 1)])
                    return _
                lax.fori_loop(0, chunk, body, 0)

    return scs(table, slen_arr)


# speedup vs baseline: 1.9243x; 1.9243x over previous
import functools

import jax
import jax.numpy as jnp
from jax import lax
from jax.experimental import pallas as pl
from jax.experimental.pallas import tpu as pltpu
from jax.experimental.pallas import tpu_sc as plsc

_CHUNK = 64


def kernel(table, seq_len):
    n, d = table.shape
    info = plsc.get_sparse_core_info()
    num_cores, num_subcores = info.num_cores, info.num_subcores
    num_workers = num_cores * num_subcores
    rows_per_w = n // num_workers
    n_chunks = rows_per_w // _CHUNK

    idx = jnp.minimum(jnp.arange(n, dtype=jnp.int32),
                      jnp.asarray(seq_len, jnp.int32) - 1)
    idx = idx.reshape(num_workers, n_chunks, _CHUNK)

    mesh = plsc.VectorSubcoreMesh(core_axis_name="c", subcore_axis_name="s")

    @functools.partial(
        pl.kernel,
        mesh=mesh,
        out_type=jax.ShapeDtypeStruct((n, d), table.dtype),
        scratch_types=[
            pltpu.VMEM((n_chunks, _CHUNK), jnp.int32),
            pltpu.VMEM((rows_per_w, d), jnp.float32),
            pltpu.SemaphoreType.DMA,
            pltpu.SemaphoreType.DMA,
        ],
    )
    def emb(table_hbm, idx_hbm, out_hbm, idx_v, rows_v, gsem, wsem):
        wid = lax.axis_index("s") * num_cores + lax.axis_index("c")
        base = wid * rows_per_w
        pltpu.sync_copy(idx_hbm.at[wid], idx_v)
        gathers = []
        for j in range(n_chunks):
            gathers.append(pltpu.async_copy(
                table_hbm.at[idx_v.at[j]],
                rows_v.at[pl.ds(j * _CHUNK, _CHUNK)],
                gsem,
            ))
        writes = []
        for j in range(n_chunks):
            gathers[j].wait()
            writes.append(pltpu.async_copy(
                rows_v.at[pl.ds(j * _CHUNK, _CHUNK)],
                out_hbm.at[pl.ds(base + j * _CHUNK, _CHUNK)],
                wsem,
            ))
        for w in writes:
            w.wait()

    return emb(table, idx)


# speedup vs baseline: 1.9261x; 1.0010x over previous
import functools

import jax
import jax.numpy as jnp
from jax import lax
from jax.experimental import pallas as pl
from jax.experimental.pallas import tpu as pltpu
from jax.experimental.pallas import tpu_sc as plsc

_CHUNK = 128


def kernel(table, seq_len):
    n, d = table.shape
    info = plsc.get_sparse_core_info()
    num_cores, num_subcores = info.num_cores, info.num_subcores
    num_workers = num_cores * num_subcores
    rows_per_w = n // num_workers
    n_chunks = rows_per_w // _CHUNK

    idx = jnp.minimum(jnp.arange(n, dtype=jnp.int32),
                      jnp.asarray(seq_len, jnp.int32) - 1)
    idx = idx.reshape(num_workers, n_chunks, _CHUNK)

    mesh = plsc.VectorSubcoreMesh(core_axis_name="c", subcore_axis_name="s")

    @functools.partial(
        pl.kernel,
        mesh=mesh,
        out_type=jax.ShapeDtypeStruct((n, d), table.dtype),
        scratch_types=[
            pltpu.VMEM((n_chunks, _CHUNK), jnp.int32),
            pltpu.VMEM((rows_per_w, d), jnp.float32),
            pltpu.SemaphoreType.DMA,
            pltpu.SemaphoreType.DMA,
        ],
    )
    def emb(table_hbm, idx_hbm, out_hbm, idx_v, rows_v, gsem, wsem):
        wid = lax.axis_index("s") * num_cores + lax.axis_index("c")
        base = wid * rows_per_w
        pltpu.sync_copy(idx_hbm.at[wid], idx_v)
        gathers = []
        for j in range(n_chunks):
            gathers.append(pltpu.async_copy(
                table_hbm.at[idx_v.at[j]],
                rows_v.at[pl.ds(j * _CHUNK, _CHUNK)],
                gsem,
            ))
        writes = []
        for j in range(n_chunks):
            gathers[j].wait()
            writes.append(pltpu.async_copy(
                rows_v.at[pl.ds(j * _CHUNK, _CHUNK)],
                out_hbm.at[pl.ds(base + j * _CHUNK, _CHUNK)],
                wsem,
            ))
        for w in writes:
            w.wait()

    return emb(table, idx)


# speedup vs baseline: 1.9344x; 1.0043x over previous
import functools

import jax
import jax.numpy as jnp
from jax import lax
from jax.experimental import pallas as pl
from jax.experimental.pallas import tpu as pltpu
from jax.experimental.pallas import tpu_sc as plsc

_CHUNK = 64


def kernel(table, seq_len):
    n, d = table.shape
    info = plsc.get_sparse_core_info()
    num_cores, num_subcores = info.num_cores, info.num_subcores
    num_workers = num_cores * num_subcores
    rows_per_w = n // num_workers
    n_chunks = rows_per_w // _CHUNK

    idx = jnp.minimum(jnp.arange(n, dtype=jnp.int32),
                      jnp.asarray(seq_len, jnp.int32) - 1)
    idx = idx.reshape(num_workers, n_chunks, _CHUNK)

    mesh = plsc.VectorSubcoreMesh(core_axis_name="c", subcore_axis_name="s")

    @functools.partial(
        pl.kernel,
        mesh=mesh,
        out_type=jax.ShapeDtypeStruct((n, d), table.dtype),
        scratch_types=[
            pltpu.VMEM((n_chunks, _CHUNK), jnp.int32),
            pltpu.VMEM((rows_per_w, d), jnp.float32),
            pltpu.SemaphoreType.DMA,
            pltpu.SemaphoreType.DMA,
        ],
    )
    def emb(table_hbm, idx_hbm, out_hbm, idx_v, rows_v, gsem, wsem):
        wid = lax.axis_index("s") * num_cores + lax.axis_index("c")
        base = wid * rows_per_w
        pltpu.sync_copy(idx_hbm.at[wid], idx_v)
        gathers = []
        for j in range(n_chunks):
            gathers.append(pltpu.async_copy(
                table_hbm.at[idx_v.at[j]],
                rows_v.at[pl.ds(j * _CHUNK, _CHUNK)],
                gsem,
            ))
        writes = []
        for j in range(n_chunks):
            gathers[j].wait()
            writes.append(pltpu.async_copy(
                rows_v.at[pl.ds(j * _CHUNK, _CHUNK)],
                out_hbm.at[pl.ds(base + j * _CHUNK, _CHUNK)],
                wsem,
            ))
        for w in writes:
            w.wait()

    return emb(table, idx)


# speedup vs baseline: 1.9379x; 1.0018x over previous
import functools

import jax
import jax.numpy as jnp
from jax import lax
from jax.experimental import pallas as pl
from jax.experimental.pallas import tpu as pltpu
from jax.experimental.pallas import tpu_sc as plsc

_CHUNK = 64


def kernel(table, seq_len):
    n, d = table.shape
    info = plsc.get_sparse_core_info()
    num_cores, num_subcores = info.num_cores, info.num_subcores
    num_workers = num_cores * num_subcores
    rows_per_w = n // num_workers
    n_chunks = rows_per_w // _CHUNK

    idx = jnp.minimum(jnp.arange(n, dtype=jnp.int32),
                      jnp.asarray(seq_len, jnp.int32) - 1)
    idx = idx.reshape(num_workers, n_chunks, _CHUNK)

    mesh = plsc.VectorSubcoreMesh(core_axis_name="c", subcore_axis_name="s")

    @functools.partial(
        pl.kernel,
        mesh=mesh,
        out_type=jax.ShapeDtypeStruct((n, d), table.dtype),
        scratch_types=[
            pltpu.VMEM((n_chunks, _CHUNK), jnp.int32),
            pltpu.VMEM((rows_per_w, d), jnp.float32),
            pltpu.SemaphoreType.DMA,
            pltpu.SemaphoreType.DMA,
        ],
    )
    def emb(table_hbm, idx_hbm, out_hbm, idx_v, rows_v, gsem, wsem):
        wid = lax.axis_index("c") * num_subcores + lax.axis_index("s")
        base = wid * rows_per_w
        pltpu.sync_copy(idx_hbm.at[wid], idx_v)
        gathers = []
        for j in range(n_chunks):
            gathers.append(pltpu.async_copy(
                table_hbm.at[idx_v.at[j]],
                rows_v.at[pl.ds(j * _CHUNK, _CHUNK)],
                gsem,
            ))
        writes = []
        for j in range(n_chunks):
            gathers[j].wait()
            writes.append(pltpu.async_copy(
                rows_v.at[pl.ds(j * _CHUNK, _CHUNK)],
                out_hbm.at[pl.ds(base + j * _CHUNK, _CHUNK)],
                wsem,
            ))
        for w in writes:
            w.wait()

    return emb(table, idx)
